# baseline (speedup 1.0000x reference)
import dataclasses
import functools

import jax
import jax.numpy as jnp
from jax import lax
from jax.experimental import pallas as pl
from jax.experimental.pallas import tpu as pltpu
from jax.experimental.pallas import tpu_sc as plsc

_NC = 2
_NS = 16
_NW = _NC * _NS
_K = 128
_H = 64
_RB = 5000


def _ceil_to(a, m):
    return (a + m - 1) // m * m


def _sc_mesh():
    return plsc.VectorSubcoreMesh(core_axis_name="c", subcore_axis_name="s")


def _sc_params():
    return pltpu.CompilerParams(use_tc_tiling_on_sc=False)


def _sc_degree(dst_t, acc_rows, nb):
    rows_per_sub = acc_rows // _NS
    chunk = 8

    @functools.partial(
        pl.kernel,
        out_type=jax.ShapeDtypeStruct((_NC, acc_rows, 16), jnp.float32),
        mesh=_sc_mesh(),
        compiler_params=_sc_params(),
        scratch_types=[
            pltpu.VMEM((nb, _K), jnp.int32),
            pltpu.VMEM((_K, 16), jnp.float32),
            pltpu.VMEM((128, 16), jnp.float32),
            pltpu.SemaphoreType.DMA,
            pltpu.VMEM_SHARED((acc_rows, 16), jnp.float32),
        ],
    )
    def deg_kernel(dst_hbm, out_hbm, didx, ones_v, zero_v, sem, acc):
        c = lax.axis_index("c")
        s = lax.axis_index("s")
        wid = s * _NC + c

        pltpu.sync_copy(dst_hbm.at[wid], didx)

        @pl.loop(0, _K)
        def _(r):
            ones_v[r, pl.ds(0, 16)] = jnp.full((16,), 1.0, jnp.float32)

        @pl.loop(0, 128)
        def _(r):
            zero_v[r, pl.ds(0, 16)] = jnp.zeros((16,), jnp.float32)

        base_rows = s * rows_per_sub

        @pl.loop(0, rows_per_sub // 128)
        def _(b):
            pltpu.sync_copy(zero_v, acc.at[pl.ds(base_rows + b * 128, 128)])

        plsc.subcore_barrier()

        @pl.loop(0, nb, step=chunk)
        def _(b):
            @pl.loop(0, chunk)
            def _(j):
                pltpu.async_copy(ones_v, acc.at[didx.at[b + j]], sem,
                                 add=True)

            @pl.loop(0, chunk)
            def _(j):
                pltpu.make_async_copy(ones_v, acc.at[didx.at[b + j]],
                                      sem).wait()

        plsc.subcore_barrier()
        pltpu.sync_copy(
            acc.at[pl.ds(base_rows, rows_per_sub)],
            out_hbm.at[c].at[pl.ds(base_rows, rows_per_sub)],
        )

    return deg_kernel(dst_t)


_NBUF = 8
_LOOKAHEAD = 4


def _sc_edge_pass(p, src_t, dst_t, acc_rows, nb):
    rows_per_sub = acc_rows // _NS

    @functools.partial(
        pl.kernel,
        out_type=jax.ShapeDtypeStruct((_NC, acc_rows, _H), jnp.float32),
        mesh=_sc_mesh(),
        compiler_params=_sc_params(),
        scratch_types=[
            pltpu.VMEM((nb, _K), jnp.int32),
            pltpu.VMEM((nb, _K), jnp.int32),
            [pltpu.VMEM((_K, _H), jnp.float32) for _ in range(_NBUF)],
            [pltpu.SemaphoreType.DMA for _ in range(_NBUF)],
            [pltpu.SemaphoreType.DMA for _ in range(_NBUF)],
            pltpu.VMEM_SHARED((acc_rows, _H), jnp.float32),
        ],
    )
    def edge_kernel(p_hbm, src_hbm, dst_hbm, out_hbm, sidx, didx, rows, gsem,
                    ssem, acc):
        c = lax.axis_index("c")
        s = lax.axis_index("s")
        wid = s * _NC + c

        pltpu.sync_copy(src_hbm.at[wid], sidx)
        pltpu.sync_copy(dst_hbm.at[wid], didx)

        @pl.loop(0, _K)
        def _(r):
            @pl.loop(0, _H, step=16)
            def _(j):
                rows[0][r, pl.ds(j, 16)] = jnp.zeros((16,), jnp.float32)

        base_rows = s * rows_per_sub

        @pl.loop(0, rows_per_sub // _K)
        def _(b):
            pltpu.sync_copy(rows[0], acc.at[pl.ds(base_rows + b * _K, _K)])

        for j in range(_LOOKAHEAD):
            pltpu.async_copy(p_hbm.at[sidx.at[j]], rows[j], gsem[j])

        plsc.subcore_barrier()

        @pl.loop(0, nb, step=_NBUF)
        def _(b):
            for j in range(_NBUF):
                blk = b + j
                jj = (j + _LOOKAHEAD) % _NBUF
                pltpu.make_async_copy(p_hbm.at[sidx.at[blk]], rows[j],
                                      gsem[j]).wait()
                pltpu.async_copy(rows[j], acc.at[didx.at[blk]], ssem[j],
                                 add=True)

                @pl.when(blk + _LOOKAHEAD < nb)
                def _():
                    @pl.when(blk >= _LOOKAHEAD)
                    def _():
                        pltpu.make_async_copy(
                            rows[jj], acc.at[didx.at[blk - _LOOKAHEAD]],
                            ssem[jj]).wait()
                    pltpu.async_copy(p_hbm.at[sidx.at[blk + _LOOKAHEAD]],
                                     rows[jj], gsem[jj])

        for j in range(_NBUF):
            pltpu.make_async_copy(rows[j], acc.at[didx.at[j]], ssem[j]).wait()

        plsc.subcore_barrier()
        pltpu.sync_copy(
            acc.at[pl.ds(base_rows, rows_per_sub)],
            out_hbm.at[c].at[pl.ds(base_rows, rows_per_sub)],
        )

    return edge_kernel(p, src_t, dst_t)


def _pre_body(x_ref, we_ref, be_ref, deg_ref, w0_ref, h_ref, dinv_ref, p_ref):
    h = jnp.dot(x_ref[...], we_ref[...], preferred_element_type=jnp.float32)
    h = jnp.maximum(h + be_ref[...], 0.0)
    h_ref[...] = h
    deg = 1.0 + deg_ref[0, :, 0] + deg_ref[1, :, 0]
    dinv = (1.0 / jnp.sqrt(deg))[:, None]
    dinv_ref[...] = dinv
    p_ref[...] = dinv * jnp.dot(h, w0_ref[...],
                                preferred_element_type=jnp.float32)


def _combine_body(part_ref, p_ref, h_ref, dinv_ref, cb_ref, g_ref, be_ref,
                  mu_ref, var_ref, wn_ref, hn_ref, pn_ref):
    dinv = dinv_ref[...]
    agg = part_ref[0] + part_ref[1] + p_ref[...]
    t = dinv * agg + cb_ref[...]
    inv = 1.0 / jnp.sqrt(var_ref[...] + 1e-5)
    t = (t - mu_ref[...]) * inv * g_ref[...] + be_ref[...]
    hn = jnp.maximum(t, 0.0) + h_ref[...]
    hn_ref[...] = hn
    pn_ref[...] = dinv * jnp.dot(hn, wn_ref[...],
                                 preferred_element_type=jnp.float32)


def kernel(x, edge_index, W_embed, b_embed, conv_W, conv_b, bn_gamma, bn_beta,
           bn_mean, bn_var):
    N, F_in = x.shape
    E = edge_index.shape[1]
    L = conv_W.shape[0]

    acc_rows = _ceil_to(N + 1, _NS * 128)
    padded_e = _ceil_to(E, _NW * _K * _NBUF)
    nb = padded_e // (_NW * _K)
    pad = padded_e - E

    pad_iota = jnp.arange(pad, dtype=edge_index.dtype)
    src_t = jnp.concatenate(
        [edge_index[0], pad_iota % N]
    ).reshape(_NW, nb, _K)
    dst_t = jnp.concatenate(
        [edge_index[1], N + pad_iota % (acc_rows - N)]
    ).reshape(_NW, nb, _K)

    grid = (N // _RB,)
    row_spec = pl.BlockSpec((_RB, _H), lambda i: (i, 0))
    vec_spec = pl.BlockSpec((1, _H), lambda i: (0, 0))
    part_spec = pl.BlockSpec((_NC, _RB, _H), lambda i: (0, i, 0))
    f32 = jnp.float32

    deg_part = _sc_degree(dst_t, acc_rows, nb)

    h, dinv, p = pl.pallas_call(
        _pre_body,
        grid=grid,
        in_specs=[pl.BlockSpec((_RB, F_in), lambda i: (i, 0)),
                  pl.BlockSpec((F_in, _H), lambda i: (0, 0)),
                  vec_spec,
                  pl.BlockSpec((_NC, _RB, 16), lambda i: (0, i, 0)),
                  pl.BlockSpec((_H, _H), lambda i: (0, 0))],
        out_specs=[row_spec, pl.BlockSpec((_RB, 1), lambda i: (i, 0)),
                   row_spec],
        out_shape=[jax.ShapeDtypeStruct((N, _H), f32),
                   jax.ShapeDtypeStruct((N, 1), f32),
                   jax.ShapeDtypeStruct((N, _H), f32)],
    )(x, W_embed, b_embed.reshape(1, _H), deg_part, conv_W[0])

    combine = pl.pallas_call(
        _combine_body,
        grid=grid,
        in_specs=[part_spec, row_spec, row_spec,
                  pl.BlockSpec((_RB, 1), lambda i: (i, 0)),
                  vec_spec, vec_spec, vec_spec, vec_spec, vec_spec,
                  pl.BlockSpec((_H, _H), lambda i: (0, 0))],
        out_specs=[row_spec, row_spec],
        out_shape=[jax.ShapeDtypeStruct((N, _H), f32),
                   jax.ShapeDtypeStruct((N, _H), f32)],
    )

    g2 = bn_gamma.reshape(1, _H)
    be2 = bn_beta.reshape(1, _H)
    mu2 = bn_mean.reshape(1, _H)
    var2 = bn_var.reshape(1, _H)

    for i in range(L):
        part = _sc_edge_pass(p, src_t, dst_t, acc_rows, nb)
        w_next = conv_W[(i + 1) % L]
        h, p = combine(part, p, h, dinv, conv_b[i].reshape(1, _H),
                       g2, be2, mu2, var2, w_next)
    return h


# speedup vs baseline: 1.0512x; 1.0512x over previous
import dataclasses
import functools

import jax
import jax.numpy as jnp
from jax import lax
from jax.experimental import pallas as pl
from jax.experimental.pallas import tpu as pltpu
from jax.experimental.pallas import tpu_sc as plsc

_NC = 2
_NS = 16
_NW = _NC * _NS
_K = 128
_H = 64
_RB = 5000


def _ceil_to(a, m):
    return (a + m - 1) // m * m


def _sc_mesh():
    return plsc.VectorSubcoreMesh(core_axis_name="c", subcore_axis_name="s")


def _sc_params():
    return pltpu.CompilerParams(use_tc_tiling_on_sc=False)


def _sc_degree(dst_t, acc_rows, nb):
    rows_per_sub = acc_rows // _NS
    chunk = 8

    @functools.partial(
        pl.kernel,
        out_type=jax.ShapeDtypeStruct((_NC, acc_rows, 16), jnp.float32),
        mesh=_sc_mesh(),
        compiler_params=_sc_params(),
        scratch_types=[
            pltpu.VMEM((nb, _K), jnp.int32),
            pltpu.VMEM((_K, 16), jnp.float32),
            pltpu.VMEM((128, 16), jnp.float32),
            pltpu.SemaphoreType.DMA,
            pltpu.VMEM_SHARED((acc_rows, 16), jnp.float32),
        ],
    )
    def deg_kernel(dst_hbm, out_hbm, didx, ones_v, zero_v, sem, acc):
        c = lax.axis_index("c")
        s = lax.axis_index("s")
        wid = s * _NC + c

        pltpu.sync_copy(dst_hbm.at[wid], didx)

        @pl.loop(0, _K)
        def _(r):
            ones_v[r, pl.ds(0, 16)] = jnp.full((16,), 1.0, jnp.float32)

        @pl.loop(0, 128)
        def _(r):
            zero_v[r, pl.ds(0, 16)] = jnp.zeros((16,), jnp.float32)

        base_rows = s * rows_per_sub

        @pl.loop(0, rows_per_sub // 128)
        def _(b):
            pltpu.sync_copy(zero_v, acc.at[pl.ds(base_rows + b * 128, 128)])

        plsc.subcore_barrier()

        @pl.loop(0, nb, step=chunk)
        def _(b):
            @pl.loop(0, chunk)
            def _(j):
                pltpu.async_copy(ones_v, acc.at[didx.at[b + j]], sem,
                                 add=True)

            @pl.loop(0, chunk)
            def _(j):
                pltpu.make_async_copy(ones_v, acc.at[didx.at[b + j]],
                                      sem).wait()

        plsc.subcore_barrier()
        pltpu.sync_copy(
            acc.at[pl.ds(base_rows, rows_per_sub)],
            out_hbm.at[c].at[pl.ds(base_rows, rows_per_sub)],
        )

    return deg_kernel(dst_t)


_NBUF = 4


def _sc_edge_pass(p, src_t, dst_t, acc_rows, nb):
    rows_per_sub = acc_rows // _NS

    @functools.partial(
        pl.kernel,
        out_type=jax.ShapeDtypeStruct((_NC, acc_rows, _H), jnp.float32),
        mesh=_sc_mesh(),
        compiler_params=_sc_params(),
        scratch_types=[
            pltpu.VMEM((nb, _K), jnp.int32),
            pltpu.VMEM((nb, _K), jnp.int32),
            [pltpu.VMEM((_K, _H), jnp.float32) for _ in range(_NBUF)],
            [pltpu.SemaphoreType.DMA for _ in range(_NBUF)],
            pltpu.VMEM_SHARED((acc_rows, _H), jnp.float32),
        ],
    )
    def edge_kernel(p_hbm, src_hbm, dst_hbm, out_hbm, sidx, didx, rows, gsem,
                    acc):
        c = lax.axis_index("c")
        s = lax.axis_index("s")
        wid = s * _NC + c

        pltpu.sync_copy(src_hbm.at[wid], sidx)
        pltpu.sync_copy(dst_hbm.at[wid], didx)

        @pl.loop(0, _K)
        def _(r):
            @pl.loop(0, _H, step=16)
            def _(j):
                rows[0][r, pl.ds(j, 16)] = jnp.zeros((16,), jnp.float32)

        base_rows = s * rows_per_sub

        @pl.loop(0, rows_per_sub // _K)
        def _(b):
            pltpu.sync_copy(rows[0], acc.at[pl.ds(base_rows + b * _K, _K)])

        for j in range(_NBUF):
            pltpu.async_copy(p_hbm.at[sidx.at[j]], rows[j], gsem[j])

        plsc.subcore_barrier()

        @pl.loop(0, nb, step=_NBUF)
        def _(b):
            for j in range(_NBUF):
                blk = b + j
                pltpu.make_async_copy(p_hbm.at[sidx.at[blk]], rows[j],
                                      gsem[j]).wait()
                pltpu.sync_copy(rows[j], acc.at[didx.at[blk]], add=True)

                @pl.when(blk + _NBUF < nb)
                def _():
                    pltpu.async_copy(p_hbm.at[sidx.at[blk + _NBUF]], rows[j],
                                     gsem[j])

        plsc.subcore_barrier()
        pltpu.sync_copy(
            acc.at[pl.ds(base_rows, rows_per_sub)],
            out_hbm.at[c].at[pl.ds(base_rows, rows_per_sub)],
        )

    return edge_kernel(p, src_t, dst_t)


def _pre_body(x_ref, we_ref, be_ref, deg_ref, w0_ref, h_ref, dinv_ref, p_ref):
    h = jnp.dot(x_ref[...], we_ref[...], preferred_element_type=jnp.float32)
    h = jnp.maximum(h + be_ref[...], 0.0)
    h_ref[...] = h
    deg = 1.0 + deg_ref[0, :, 0] + deg_ref[1, :, 0]
    dinv = (1.0 / jnp.sqrt(deg))[:, None]
    dinv_ref[...] = dinv
    p_ref[...] = dinv * jnp.dot(h, w0_ref[...],
                                preferred_element_type=jnp.float32)


def _combine_body(part_ref, p_ref, h_ref, dinv_ref, cb_ref, g_ref, be_ref,
                  mu_ref, var_ref, wn_ref, hn_ref, pn_ref):
    dinv = dinv_ref[...]
    agg = part_ref[0] + part_ref[1] + p_ref[...]
    t = dinv * agg + cb_ref[...]
    inv = 1.0 / jnp.sqrt(var_ref[...] + 1e-5)
    t = (t - mu_ref[...]) * inv * g_ref[...] + be_ref[...]
    hn = jnp.maximum(t, 0.0) + h_ref[...]
    hn_ref[...] = hn
    pn_ref[...] = dinv * jnp.dot(hn, wn_ref[...],
                                 preferred_element_type=jnp.float32)


def kernel(x, edge_index, W_embed, b_embed, conv_W, conv_b, bn_gamma, bn_beta,
           bn_mean, bn_var):
    N, F_in = x.shape
    E = edge_index.shape[1]
    L = conv_W.shape[0]

    acc_rows = _ceil_to(N + 1, _NS * 128)
    padded_e = _ceil_to(E, _NW * _K * _NBUF)
    nb = padded_e // (_NW * _K)
    pad = padded_e - E

    pad_iota = jnp.arange(pad, dtype=edge_index.dtype)
    src_t = jnp.concatenate(
        [edge_index[0], pad_iota % N]
    ).reshape(_NW, nb, _K)
    dst_t = jnp.concatenate(
        [edge_index[1], N + pad_iota % (acc_rows - N)]
    ).reshape(_NW, nb, _K)

    grid = (N // _RB,)
    row_spec = pl.BlockSpec((_RB, _H), lambda i: (i, 0))
    vec_spec = pl.BlockSpec((1, _H), lambda i: (0, 0))
    part_spec = pl.BlockSpec((_NC, _RB, _H), lambda i: (0, i, 0))
    f32 = jnp.float32

    deg_part = _sc_degree(dst_t, acc_rows, nb)

    h, dinv, p = pl.pallas_call(
        _pre_body,
        grid=grid,
        in_specs=[pl.BlockSpec((_RB, F_in), lambda i: (i, 0)),
                  pl.BlockSpec((F_in, _H), lambda i: (0, 0)),
                  vec_spec,
                  pl.BlockSpec((_NC, _RB, 16), lambda i: (0, i, 0)),
                  pl.BlockSpec((_H, _H), lambda i: (0, 0))],
        out_specs=[row_spec, pl.BlockSpec((_RB, 1), lambda i: (i, 0)),
                   row_spec],
        out_shape=[jax.ShapeDtypeStruct((N, _H), f32),
                   jax.ShapeDtypeStruct((N, 1), f32),
                   jax.ShapeDtypeStruct((N, _H), f32)],
    )(x, W_embed, b_embed.reshape(1, _H), deg_part, conv_W[0])

    combine = pl.pallas_call(
        _combine_body,
        grid=grid,
        in_specs=[part_spec, row_spec, row_spec,
                  pl.BlockSpec((_RB, 1), lambda i: (i, 0)),
                  vec_spec, vec_spec, vec_spec, vec_spec, vec_spec,
                  pl.BlockSpec((_H, _H), lambda i: (0, 0))],
        out_specs=[row_spec, row_spec],
        out_shape=[jax.ShapeDtypeStruct((N, _H), f32),
                   jax.ShapeDtypeStruct((N, _H), f32)],
    )

    g2 = bn_gamma.reshape(1, _H)
    be2 = bn_beta.reshape(1, _H)
    mu2 = bn_mean.reshape(1, _H)
    var2 = bn_var.reshape(1, _H)

    for i in range(L):
        part = _sc_edge_pass(p, src_t, dst_t, acc_rows, nb)
        w_next = conv_W[(i + 1) % L]
        h, p = combine(part, p, h, dinv, conv_b[i].reshape(1, _H),
                       g2, be2, mu2, var2, w_next)
    return h


# speedup vs baseline: 1.0856x; 1.0328x over previous
import dataclasses
import functools

import jax
import jax.numpy as jnp
from jax import lax
from jax.experimental import pallas as pl
from jax.experimental.pallas import tpu as pltpu
from jax.experimental.pallas import tpu_sc as plsc

_NC = 2
_NS = 16
_NW = _NC * _NS
_K = 128
_H = 64
_RB = 5000


def _ceil_to(a, m):
    return (a + m - 1) // m * m


def _sc_mesh():
    return plsc.VectorSubcoreMesh(core_axis_name="c", subcore_axis_name="s")


def _sc_params():
    return pltpu.CompilerParams(use_tc_tiling_on_sc=False)


def _sc_degree(edge3, acc_rows, nb, k):
    rows_per_sub = acc_rows // _NS
    chunk = 8

    @functools.partial(
        pl.kernel,
        out_type=jax.ShapeDtypeStruct((_NC, acc_rows, 16), jnp.float32),
        mesh=_sc_mesh(),
        compiler_params=_sc_params(),
        scratch_types=[
            pltpu.VMEM((nb, k), jnp.int32),
            pltpu.VMEM((k, 16), jnp.float32),
            pltpu.VMEM((128, 16), jnp.float32),
            pltpu.SemaphoreType.DMA,
            pltpu.VMEM_SHARED((acc_rows, 16), jnp.float32),
        ],
    )
    def deg_kernel(edge_hbm, out_hbm, didx, ones_v, zero_v, sem, acc):
        c = lax.axis_index("c")
        s = lax.axis_index("s")
        wid = s * _NC + c

        pltpu.sync_copy(edge_hbm.at[1].at[pl.ds(wid * nb, nb)], didx)

        @pl.loop(0, k)
        def _(r):
            ones_v[r, pl.ds(0, 16)] = jnp.full((16,), 1.0, jnp.float32)

        @pl.loop(0, 128)
        def _(r):
            zero_v[r, pl.ds(0, 16)] = jnp.zeros((16,), jnp.float32)

        base_rows = s * rows_per_sub

        @pl.loop(0, rows_per_sub // 128)
        def _(b):
            pltpu.sync_copy(zero_v, acc.at[pl.ds(base_rows + b * 128, 128)])

        plsc.subcore_barrier()

        @pl.loop(0, nb, step=chunk)
        def _(b):
            @pl.loop(0, chunk)
            def _(j):
                pltpu.async_copy(ones_v, acc.at[didx.at[b + j]], sem,
                                 add=True)

            @pl.loop(0, chunk)
            def _(j):
                pltpu.make_async_copy(ones_v, acc.at[didx.at[b + j]],
                                      sem).wait()

        plsc.subcore_barrier()
        pltpu.sync_copy(
            acc.at[pl.ds(base_rows, rows_per_sub)],
            out_hbm.at[c].at[pl.ds(base_rows, rows_per_sub)],
        )

    return deg_kernel(edge3)


_NBUF = 4


def _sc_edge_pass(p, edge3, acc_rows, nb, k):
    rows_per_sub = acc_rows // _NS
    nfull = rows_per_sub // k
    nrem = rows_per_sub - nfull * k

    @functools.partial(
        pl.kernel,
        out_type=jax.ShapeDtypeStruct((_NC, acc_rows, _H), jnp.float32),
        mesh=_sc_mesh(),
        compiler_params=_sc_params(),
        scratch_types=[
            pltpu.VMEM((nb, k), jnp.int32),
            pltpu.VMEM((nb, k), jnp.int32),
            [pltpu.VMEM((k, _H), jnp.float32) for _ in range(_NBUF)],
            [pltpu.SemaphoreType.DMA for _ in range(_NBUF)],
            pltpu.VMEM_SHARED((acc_rows, _H), jnp.float32),
        ],
    )
    def edge_kernel(p_hbm, edge_hbm, out_hbm, sidx, didx, rows, gsem,
                    acc):
        c = lax.axis_index("c")
        s = lax.axis_index("s")
        wid = s * _NC + c

        pltpu.sync_copy(edge_hbm.at[0].at[pl.ds(wid * nb, nb)], sidx)
        pltpu.sync_copy(edge_hbm.at[1].at[pl.ds(wid * nb, nb)], didx)

        @pl.loop(0, k)
        def _(r):
            @pl.loop(0, _H, step=16)
            def _(j):
                rows[0][r, pl.ds(j, 16)] = jnp.zeros((16,), jnp.float32)

        base_rows = s * rows_per_sub

        @pl.loop(0, nfull)
        def _(b):
            pltpu.sync_copy(rows[0], acc.at[pl.ds(base_rows + b * k, k)])

        if nrem:
            pltpu.sync_copy(
                rows[0].at[pl.ds(0, nrem)],
                acc.at[pl.ds(base_rows + nfull * k, nrem)])

        for j in range(_NBUF):
            pltpu.async_copy(p_hbm.at[sidx.at[j]], rows[j], gsem[j])

        plsc.subcore_barrier()

        @pl.loop(0, nb, step=_NBUF)
        def _(b):
            for j in range(_NBUF):
                blk = b + j
                pltpu.make_async_copy(p_hbm.at[sidx.at[blk]], rows[j],
                                      gsem[j]).wait()
                pltpu.sync_copy(rows[j], acc.at[didx.at[blk]], add=True)

                @pl.when(blk + _NBUF < nb)
                def _():
                    pltpu.async_copy(p_hbm.at[sidx.at[blk + _NBUF]], rows[j],
                                     gsem[j])

        plsc.subcore_barrier()
        pltpu.sync_copy(
            acc.at[pl.ds(base_rows, rows_per_sub)],
            out_hbm.at[c].at[pl.ds(base_rows, rows_per_sub)],
        )

    return edge_kernel(p, edge3)


def _pre_body(x_ref, we_ref, be_ref, deg_ref, w0_ref, h_ref, dinv_ref, p_ref):
    h = jnp.dot(x_ref[...], we_ref[...], preferred_element_type=jnp.float32)
    h = jnp.maximum(h + be_ref[...], 0.0)
    h_ref[...] = h
    deg = 1.0 + deg_ref[0, :, 0] + deg_ref[1, :, 0]
    dinv = (1.0 / jnp.sqrt(deg))[:, None]
    dinv_ref[...] = dinv
    p_ref[...] = dinv * jnp.dot(h, w0_ref[...],
                                preferred_element_type=jnp.float32)


def _combine_body(part_ref, p_ref, h_ref, dinv_ref, cb_ref, g_ref, be_ref,
                  mu_ref, var_ref, wn_ref, hn_ref, pn_ref):
    dinv = dinv_ref[...]
    agg = part_ref[0] + part_ref[1] + p_ref[...]
    t = dinv * agg + cb_ref[...]
    inv = 1.0 / jnp.sqrt(var_ref[...] + 1e-5)
    t = (t - mu_ref[...]) * inv * g_ref[...] + be_ref[...]
    hn = jnp.maximum(t, 0.0) + h_ref[...]
    hn_ref[...] = hn
    pn_ref[...] = dinv * jnp.dot(hn, wn_ref[...],
                                 preferred_element_type=jnp.float32)


def kernel(x, edge_index, W_embed, b_embed, conv_W, conv_b, bn_gamma, bn_beta,
           bn_mean, bn_var):
    N, F_in = x.shape
    E = edge_index.shape[1]
    L = conv_W.shape[0]

    acc_rows = _ceil_to(N + 1, _NS * 128)
    e_per_w = E // _NW
    k = next(kk for kk in range(128, 0, -1)
             if e_per_w % (kk * _NBUF) == 0)
    nb = e_per_w // k
    edge3 = edge_index.reshape(2, E // k, k)

    grid = (N // _RB,)
    row_spec = pl.BlockSpec((_RB, _H), lambda i: (i, 0))
    vec_spec = pl.BlockSpec((1, _H), lambda i: (0, 0))
    part_spec = pl.BlockSpec((_NC, _RB, _H), lambda i: (0, i, 0))
    f32 = jnp.float32

    deg_part = _sc_degree(edge3, acc_rows, nb, k)

    h, dinv, p = pl.pallas_call(
        _pre_body,
        grid=grid,
        in_specs=[pl.BlockSpec((_RB, F_in), lambda i: (i, 0)),
                  pl.BlockSpec((F_in, _H), lambda i: (0, 0)),
                  vec_spec,
                  pl.BlockSpec((_NC, _RB, 16), lambda i: (0, i, 0)),
                  pl.BlockSpec((_H, _H), lambda i: (0, 0))],
        out_specs=[row_spec, pl.BlockSpec((_RB, 1), lambda i: (i, 0)),
                   row_spec],
        out_shape=[jax.ShapeDtypeStruct((N, _H), f32),
                   jax.ShapeDtypeStruct((N, 1), f32),
                   jax.ShapeDtypeStruct((N, _H), f32)],
    )(x, W_embed, b_embed.reshape(1, _H), deg_part, conv_W[0])

    combine = pl.pallas_call(
        _combine_body,
        grid=grid,
        in_specs=[part_spec, row_spec, row_spec,
                  pl.BlockSpec((_RB, 1), lambda i: (i, 0)),
                  vec_spec, vec_spec, vec_spec, vec_spec, vec_spec,
                  pl.BlockSpec((_H, _H), lambda i: (0, 0))],
        out_specs=[row_spec, row_spec],
        out_shape=[jax.ShapeDtypeStruct((N, _H), f32),
                   jax.ShapeDtypeStruct((N, _H), f32)],
    )

    g2 = bn_gamma.reshape(1, _H)
    be2 = bn_beta.reshape(1, _H)
    mu2 = bn_mean.reshape(1, _H)
    var2 = bn_var.reshape(1, _H)

    for i in range(L):
        part = _sc_edge_pass(p, edge3, acc_rows, nb, k)
        w_next = conv_W[(i + 1) % L]
        h, p = combine(part, p, h, dinv, conv_b[i].reshape(1, _H),
                       g2, be2, mu2, var2, w_next)
    return h


# speedup vs baseline: 1.3111x; 1.2077x over previous
import dataclasses
import functools

import jax
import jax.numpy as jnp
from jax import lax
from jax.experimental import pallas as pl
from jax.experimental.pallas import tpu as pltpu
from jax.experimental.pallas import tpu_sc as plsc

_NC = 2
_NS = 16
_NW = _NC * _NS
_K = 128
_H = 64
_RB = 5000


def _ceil_to(a, m):
    return (a + m - 1) // m * m


def _sc_mesh():
    return plsc.VectorSubcoreMesh(core_axis_name="c", subcore_axis_name="s")


def _sc_params():
    return pltpu.CompilerParams(use_tc_tiling_on_sc=False)


def _sc_degree(edge3, acc_rows, nb, k):
    rows_per_sub = acc_rows // _NS
    chunk = 8

    @functools.partial(
        pl.kernel,
        out_type=jax.ShapeDtypeStruct((_NC, acc_rows, _H), jnp.float32),
        mesh=_sc_mesh(),
        compiler_params=_sc_params(),
        scratch_types=[
            pltpu.VMEM((nb, k), jnp.int32),
            pltpu.VMEM((k, 16), jnp.float32),
            pltpu.VMEM((128, 16), jnp.float32),
            pltpu.SemaphoreType.DMA,
            pltpu.VMEM_SHARED((acc_rows, 16), jnp.float32),
        ],
    )
    def deg_kernel(edge_hbm, out_hbm, didx, ones_v, zero_v, sem, acc):
        c = lax.axis_index("c")
        s = lax.axis_index("s")
        wid = s * _NC + c

        pltpu.sync_copy(edge_hbm.at[1].at[pl.ds(wid * nb, nb)], didx)

        @pl.loop(0, k)
        def _(r):
            ones_v[r, pl.ds(0, 16)] = jnp.full((16,), 1.0, jnp.float32)

        @pl.loop(0, 128)
        def _(r):
            zero_v[r, pl.ds(0, 16)] = jnp.zeros((16,), jnp.float32)

        base_rows = s * rows_per_sub

        @pl.loop(0, rows_per_sub // 128)
        def _(b):
            pltpu.sync_copy(zero_v, acc.at[pl.ds(base_rows + b * 128, 128)])

        plsc.subcore_barrier()

        @pl.loop(0, nb, step=chunk)
        def _(b):
            @pl.loop(0, chunk)
            def _(j):
                pltpu.async_copy(ones_v, acc.at[didx.at[b + j]], sem,
                                 add=True)

            @pl.loop(0, chunk)
            def _(j):
                pltpu.make_async_copy(ones_v, acc.at[didx.at[b + j]],
                                      sem).wait()

        plsc.subcore_barrier()
        for j in range(_H // 16):
            pltpu.sync_copy(
                acc.at[pl.ds(base_rows, rows_per_sub)],
                out_hbm.at[c].at[pl.ds(base_rows, rows_per_sub),
                                 pl.ds(16 * j, 16)],
            )

    return deg_kernel(edge3)


_NBUF = 4


def _sc_edge_pass(p, edge3, acc_rows, nb, k):
    rows_per_sub = acc_rows // _NS
    nfull = rows_per_sub // k
    nrem = rows_per_sub - nfull * k

    @functools.partial(
        pl.kernel,
        out_type=jax.ShapeDtypeStruct((_NC, acc_rows, _H), jnp.float32),
        mesh=_sc_mesh(),
        compiler_params=_sc_params(),
        scratch_types=[
            pltpu.VMEM((nb, k), jnp.int32),
            pltpu.VMEM((nb, k), jnp.int32),
            [pltpu.VMEM((k, _H), jnp.float32) for _ in range(_NBUF)],
            [pltpu.SemaphoreType.DMA for _ in range(_NBUF)],
            pltpu.VMEM_SHARED((acc_rows, _H), jnp.float32),
        ],
    )
    def edge_kernel(p_hbm, edge_hbm, out_hbm, sidx, didx, rows, gsem,
                    acc):
        c = lax.axis_index("c")
        s = lax.axis_index("s")
        wid = s * _NC + c

        pltpu.sync_copy(edge_hbm.at[0].at[pl.ds(wid * nb, nb)], sidx)
        pltpu.sync_copy(edge_hbm.at[1].at[pl.ds(wid * nb, nb)], didx)

        @pl.loop(0, k)
        def _(r):
            @pl.loop(0, _H, step=16)
            def _(j):
                rows[0][r, pl.ds(j, 16)] = jnp.zeros((16,), jnp.float32)

        base_rows = s * rows_per_sub

        @pl.loop(0, nfull)
        def _(b):
            pltpu.sync_copy(rows[0], acc.at[pl.ds(base_rows + b * k, k)])

        if nrem:
            pltpu.sync_copy(
                rows[0].at[pl.ds(0, nrem)],
                acc.at[pl.ds(base_rows + nfull * k, nrem)])

        for j in range(_NBUF):
            pltpu.async_copy(p_hbm.at[sidx.at[j]], rows[j], gsem[j])

        plsc.subcore_barrier()

        @pl.loop(0, nb, step=_NBUF)
        def _(b):
            for j in range(_NBUF):
                blk = b + j
                pltpu.make_async_copy(p_hbm.at[sidx.at[blk]], rows[j],
                                      gsem[j]).wait()
                pltpu.sync_copy(rows[j], acc.at[didx.at[blk]], add=True)

                @pl.when(blk + _NBUF < nb)
                def _():
                    pltpu.async_copy(p_hbm.at[sidx.at[blk + _NBUF]], rows[j],
                                     gsem[j])

        plsc.subcore_barrier()
        pltpu.sync_copy(
            acc.at[pl.ds(base_rows, rows_per_sub)],
            out_hbm.at[c].at[pl.ds(base_rows, rows_per_sub)],
        )

    return edge_kernel(p, edge3)


def _embed_body(xf_ref, wef_ref, bef_ref, hf_ref):
    hf = jnp.dot(xf_ref[...], wef_ref[...], preferred_element_type=jnp.float32)
    hf_ref[...] = jnp.maximum(hf + bef_ref[...], 0.0)


def _p0_body(deg_ref, hf_ref, w0_ref, dinvf_ref, pf_ref):
    nf = hf_ref.shape[0]
    degf = deg_ref[0, :nf, :] + deg_ref[1, :nf, :]
    dinvf = 1.0 / jnp.sqrt(1.0 + degf)
    dinvf_ref[...] = dinvf
    pf_ref[...] = dinvf * jnp.dot(hf_ref[...], w0_ref[...],
                                  preferred_element_type=jnp.float32)


def _combine_body(part_ref, pf_ref, hf_ref, dinvf_ref, cb_ref, g_ref, be_ref,
                  mu_ref, var_ref, wn_ref, hn_ref, pn_ref):
    nf = pf_ref.shape[0]
    dinvf = dinvf_ref[...]
    aggf = part_ref[0, :nf, :] + part_ref[1, :nf, :] + pf_ref[...]
    t = dinvf * aggf + cb_ref[...]
    inv = 1.0 / jnp.sqrt(var_ref[...] + 1e-5)
    t = (t - mu_ref[...]) * inv * g_ref[...] + be_ref[...]
    hn = jnp.maximum(t, 0.0) + hf_ref[...]
    hn_ref[...] = hn
    pn_ref[...] = dinvf * jnp.dot(hn, wn_ref[...],
                                  preferred_element_type=jnp.float32)


def kernel(x, edge_index, W_embed, b_embed, conv_W, conv_b, bn_gamma, bn_beta,
           bn_mean, bn_var):
    N, F_in = x.shape
    E = edge_index.shape[1]
    L = conv_W.shape[0]

    acc_rows = _ceil_to(N + 1, _NS * 128)
    e_per_w = E // _NW
    k = next(kk for kk in range(128, 0, -1)
             if e_per_w % (kk * _NBUF) == 0)
    nb = e_per_w // k
    edge3 = edge_index.reshape(2, E // k, k)

    nf = N // 2
    fb = 1000
    foldr_spec = pl.BlockSpec((nf, 128), lambda: (0, 0))
    foldb_spec = pl.BlockSpec((fb, 128), lambda i: (i, 0))
    vec_spec = pl.BlockSpec((1, 128), lambda: (0, 0))
    vecb_spec = pl.BlockSpec((1, 128), lambda i: (0, 0))
    wd_spec = pl.BlockSpec((128, 128), lambda: (0, 0))
    wdb_spec = pl.BlockSpec((128, 128), lambda i: (0, 0))
    f32 = jnp.float32

    def tile2(v):
        return jnp.concatenate([v, v]).reshape(1, 128)

    eye2 = jnp.eye(2, dtype=f32)
    conv_Wd = jnp.kron(eye2, conv_W)
    W_embed_d = jnp.kron(eye2, W_embed)
    xf = x.reshape(nf, 2 * F_in)

    deg_part = _sc_degree(edge3, acc_rows, nb, k)
    deg_f = deg_part.reshape(_NC, acc_rows // 2, 128)

    hf = pl.pallas_call(
        _embed_body,
        in_specs=[pl.BlockSpec((nf, 2 * F_in), lambda: (0, 0)),
                  pl.BlockSpec((2 * F_in, 128), lambda: (0, 0)),
                  vec_spec],
        out_specs=foldr_spec,
        out_shape=jax.ShapeDtypeStruct((nf, 128), f32),
    )(xf, W_embed_d, tile2(b_embed))

    dinvf, pf = pl.pallas_call(
        _p0_body,
        grid=(nf // fb,),
        in_specs=[pl.BlockSpec((_NC, fb, 128), lambda i: (0, i, 0)),
                  foldb_spec, wdb_spec],
        out_specs=[foldb_spec, foldb_spec],
        out_shape=[jax.ShapeDtypeStruct((nf, 128), f32),
                   jax.ShapeDtypeStruct((nf, 128), f32)],
    )(deg_f, hf, conv_Wd[0])

    combine = pl.pallas_call(
        _combine_body,
        grid=(nf // fb,),
        in_specs=[pl.BlockSpec((_NC, fb, 128), lambda i: (0, i, 0)),
                  foldb_spec, foldb_spec, foldb_spec,
                  vecb_spec, vecb_spec, vecb_spec, vecb_spec, vecb_spec,
                  wdb_spec],
        out_specs=[foldb_spec, foldb_spec],
        out_shape=[jax.ShapeDtypeStruct((nf, 128), f32),
                   jax.ShapeDtypeStruct((nf, 128), f32)],
    )

    g2 = tile2(bn_gamma)
    be2 = tile2(bn_beta)
    mu2 = tile2(bn_mean)
    var2 = tile2(bn_var)

    for i in range(L):
        part = _sc_edge_pass(pf.reshape(N, _H), edge3, acc_rows, nb, k)
        part_f = part.reshape(_NC, acc_rows // 2, 128)
        hf, pf = combine(part_f, pf, hf, dinvf, tile2(conv_b[i]),
                         g2, be2, mu2, var2, conv_Wd[(i + 1) % L])
    return hf.reshape(N, _H)


# speedup vs baseline: 1.3222x; 1.0084x over previous
import dataclasses
import functools

import jax
import jax.numpy as jnp
from jax import lax
from jax.experimental import pallas as pl
from jax.experimental.pallas import tpu as pltpu
from jax.experimental.pallas import tpu_sc as plsc

_NC = 2
_NS = 16
_NW = _NC * _NS
_K = 128
_H = 64
_RB = 5000


def _ceil_to(a, m):
    return (a + m - 1) // m * m


def _sc_mesh():
    return plsc.VectorSubcoreMesh(core_axis_name="c", subcore_axis_name="s")


def _sc_params():
    return pltpu.CompilerParams(use_tc_tiling_on_sc=False)


def _sc_degree(edge3, acc_rows, nb, k):
    rows_per_sub = acc_rows // _NS
    chunk = 8

    @functools.partial(
        pl.kernel,
        out_type=jax.ShapeDtypeStruct((_NC, acc_rows, _H), jnp.float32),
        mesh=_sc_mesh(),
        compiler_params=_sc_params(),
        scratch_types=[
            pltpu.VMEM((nb, k), jnp.int32),
            pltpu.VMEM((k, 16), jnp.float32),
            pltpu.VMEM((128, 16), jnp.float32),
            pltpu.SemaphoreType.DMA,
            pltpu.VMEM_SHARED((acc_rows, 16), jnp.float32),
        ],
    )
    def deg_kernel(edge_hbm, out_hbm, didx, ones_v, zero_v, sem, acc):
        c = lax.axis_index("c")
        s = lax.axis_index("s")
        wid = s * _NC + c

        pltpu.sync_copy(edge_hbm.at[1].at[pl.ds(wid * nb, nb)], didx)

        @pl.loop(0, k)
        def _(r):
            ones_v[r, pl.ds(0, 16)] = jnp.full((16,), 1.0, jnp.float32)

        @pl.loop(0, 128)
        def _(r):
            zero_v[r, pl.ds(0, 16)] = jnp.zeros((16,), jnp.float32)

        base_rows = s * rows_per_sub

        @pl.loop(0, rows_per_sub // 128)
        def _(b):
            pltpu.sync_copy(zero_v, acc.at[pl.ds(base_rows + b * 128, 128)])

        plsc.subcore_barrier()

        @pl.loop(0, nb, step=chunk)
        def _(b):
            @pl.loop(0, chunk)
            def _(j):
                pltpu.async_copy(ones_v, acc.at[didx.at[b + j]], sem,
                                 add=True)

            @pl.loop(0, chunk)
            def _(j):
                pltpu.make_async_copy(ones_v, acc.at[didx.at[b + j]],
                                      sem).wait()

        plsc.subcore_barrier()
        for j in range(_H // 16):
            pltpu.sync_copy(
                acc.at[pl.ds(base_rows, rows_per_sub)],
                out_hbm.at[c].at[pl.ds(base_rows, rows_per_sub),
                                 pl.ds(16 * j, 16)],
            )

    return deg_kernel(edge3)


_NBUF = 4


def _sc_edge_pass(p, edge3, acc_rows, nb, k):
    rows_per_sub = acc_rows // _NS
    nfull = rows_per_sub // k
    nrem = rows_per_sub - nfull * k

    @functools.partial(
        pl.kernel,
        out_type=jax.ShapeDtypeStruct((_NC, acc_rows, _H), jnp.float32),
        mesh=_sc_mesh(),
        compiler_params=_sc_params(),
        scratch_types=[
            pltpu.VMEM((nb, k), jnp.int32),
            pltpu.VMEM((nb, k), jnp.int32),
            [pltpu.VMEM((k, _H), jnp.float32) for _ in range(_NBUF)],
            [pltpu.SemaphoreType.DMA for _ in range(_NBUF)],
            pltpu.VMEM_SHARED((acc_rows, _H), jnp.float32),
        ],
    )
    def edge_kernel(p_hbm, edge_hbm, out_hbm, sidx, didx, rows, gsem,
                    acc):
        c = lax.axis_index("c")
        s = lax.axis_index("s")
        wid = s * _NC + c

        pltpu.sync_copy(edge_hbm.at[0].at[pl.ds(wid * nb, nb)], sidx)
        pltpu.sync_copy(edge_hbm.at[1].at[pl.ds(wid * nb, nb)], didx)

        @pl.loop(0, k)
        def _(r):
            @pl.loop(0, _H, step=16)
            def _(j):
                rows[0][r, pl.ds(j, 16)] = jnp.zeros((16,), jnp.float32)

        base_rows = s * rows_per_sub

        @pl.loop(0, nfull)
        def _(b):
            pltpu.sync_copy(rows[0], acc.at[pl.ds(base_rows + b * k, k)])

        if nrem:
            pltpu.sync_copy(
                rows[0].at[pl.ds(0, nrem)],
                acc.at[pl.ds(base_rows + nfull * k, nrem)])

        for j in range(_NBUF):
            pltpu.async_copy(p_hbm.at[sidx.at[j]], rows[j], gsem[j])

        plsc.subcore_barrier()

        @pl.loop(0, nb, step=_NBUF)
        def _(b):
            for j in range(_NBUF):
                blk = b + j
                pltpu.make_async_copy(p_hbm.at[sidx.at[blk]], rows[j],
                                      gsem[j]).wait()
                pltpu.sync_copy(rows[j], acc.at[didx.at[blk]], add=True)

                @pl.when(blk + _NBUF < nb)
                def _():
                    pltpu.async_copy(p_hbm.at[sidx.at[blk + _NBUF]], rows[j],
                                     gsem[j])

        plsc.subcore_barrier()
        pltpu.sync_copy(
            acc.at[pl.ds(base_rows, rows_per_sub)],
            out_hbm.at[c].at[pl.ds(base_rows, rows_per_sub)],
        )

    return edge_kernel(p, edge3)


def _embed_body(xf_ref, wef_ref, bef_ref, hf_ref):
    hf = jnp.dot(xf_ref[...], wef_ref[...], preferred_element_type=jnp.float32)
    hf_ref[...] = jnp.maximum(hf + bef_ref[...], 0.0)


def _p0_body(deg_ref, hf_ref, w0_ref, dinvf_ref, pf_ref):
    nf = hf_ref.shape[0]
    degf = deg_ref[0, :nf, :] + deg_ref[1, :nf, :]
    dinvf = 1.0 / jnp.sqrt(1.0 + degf)
    dinvf_ref[...] = dinvf
    pf_ref[...] = dinvf * jnp.dot(hf_ref[...], w0_ref[...],
                                  preferred_element_type=jnp.float32)


def _combine_body(part_ref, pf_ref, hf_ref, dinvf_ref, cb_ref, g_ref, be_ref,
                  mu_ref, var_ref, wn_ref, hn_ref, pn_ref):
    nf = pf_ref.shape[0]
    dinvf = dinvf_ref[...]
    aggf = part_ref[0, :nf, :] + part_ref[1, :nf, :] + pf_ref[...]
    t = dinvf * aggf + cb_ref[...]
    inv = 1.0 / jnp.sqrt(var_ref[...] + 1e-5)
    t = (t - mu_ref[...]) * inv * g_ref[...] + be_ref[...]
    hn = jnp.maximum(t, 0.0) + hf_ref[...]
    hn_ref[...] = hn
    pn_ref[...] = dinvf * jnp.dot(hn, wn_ref[...],
                                  preferred_element_type=jnp.float32)


def kernel(x, edge_index, W_embed, b_embed, conv_W, conv_b, bn_gamma, bn_beta,
           bn_mean, bn_var):
    N, F_in = x.shape
    E = edge_index.shape[1]
    L = conv_W.shape[0]

    acc_rows = _ceil_to(N + 1, _NS * 128)
    e_per_w = E // _NW
    k = next(kk for kk in range(128, 0, -1)
             if e_per_w % (kk * _NBUF) == 0)
    nb = e_per_w // k
    edge3 = edge_index.reshape(2, E // k, k)

    nf = N // 2
    foldr_spec = pl.BlockSpec((nf, 128), lambda: (0, 0))
    vec_spec = pl.BlockSpec((1, 128), lambda: (0, 0))
    wd_spec = pl.BlockSpec((128, 128), lambda: (0, 0))
    f32 = jnp.float32

    def tile2(v):
        return jnp.concatenate([v, v]).reshape(1, 128)

    eye2 = jnp.eye(2, dtype=f32)
    conv_Wd = jnp.kron(eye2, conv_W)
    W_embed_d = jnp.kron(eye2, W_embed)
    xf = x.reshape(nf, 2 * F_in)

    deg_part = _sc_degree(edge3, acc_rows, nb, k)
    deg_f = deg_part.reshape(_NC, acc_rows // 2, 128)

    hf = pl.pallas_call(
        _embed_body,
        in_specs=[pl.BlockSpec((nf, 2 * F_in), lambda: (0, 0)),
                  pl.BlockSpec((2 * F_in, 128), lambda: (0, 0)),
                  vec_spec],
        out_specs=foldr_spec,
        out_shape=jax.ShapeDtypeStruct((nf, 128), f32),
    )(xf, W_embed_d, tile2(b_embed))

    dinvf, pf = pl.pallas_call(
        _p0_body,
        in_specs=[pl.BlockSpec((_NC, acc_rows // 2, 128), lambda: (0, 0, 0)),
                  foldr_spec, wd_spec],
        out_specs=[foldr_spec, foldr_spec],
        out_shape=[jax.ShapeDtypeStruct((nf, 128), f32),
                   jax.ShapeDtypeStruct((nf, 128), f32)],
    )(deg_f, hf, conv_Wd[0])

    combine = pl.pallas_call(
        _combine_body,
        in_specs=[pl.BlockSpec((_NC, acc_rows // 2, 128),
                               lambda: (0, 0, 0)),
                  foldr_spec, foldr_spec, foldr_spec,
                  vec_spec, vec_spec, vec_spec, vec_spec, vec_spec,
                  wd_spec],
        out_specs=[foldr_spec, foldr_spec],
        out_shape=[jax.ShapeDtypeStruct((nf, 128), f32),
                   jax.ShapeDtypeStruct((nf, 128), f32)],
    )

    g2 = tile2(bn_gamma)
    be2 = tile2(bn_beta)
    mu2 = tile2(bn_mean)
    var2 = tile2(bn_var)

    for i in range(L):
        part = _sc_edge_pass(pf.reshape(N, _H), edge3, acc_rows, nb, k)
        part_f = part.reshape(_NC, acc_rows // 2, 128)
        hf, pf = combine(part_f, pf, hf, dinvf, tile2(conv_b[i]),
                         g2, be2, mu2, var2, conv_Wd[(i + 1) % L])
    return hf.reshape(N, _H)
